# baseline (speedup 1.0000x reference)
import functools

import jax
import jax.numpy as jnp
from jax import lax
from jax.experimental import pallas as pl
from jax.experimental.pallas import tpu as pltpu
from jax.experimental.pallas import tpu_sc as plsc

NC = 2
NS = 16
G = 256


def _make_sc_agg(N, E, D, K):
    W = NC * NS
    epw = E // W
    nchunk = epw // K
    npass = 5
    ppc = nchunk // npass
    NBUF = 4
    nper = ((N // NS) // 8) * 8
    ntail = N - NS * nper
    mesh = plsc.VectorSubcoreMesh(core_axis_name="c", subcore_axis_name="s")

    @functools.partial(
        pl.kernel,
        out_type=jax.ShapeDtypeStruct((NC, N, D), jnp.float32),
        mesh=mesh,
        scratch_types=[
            [pltpu.VMEM((ppc, K), jnp.int32) for _ in range(2)],
            [pltpu.VMEM((ppc, K), jnp.int32) for _ in range(2)],
            [pltpu.VMEM((K, D), jnp.float32) for _ in range(NBUF)],
            pltpu.VMEM_SHARED((N, D), jnp.float32),
            [pltpu.SemaphoreType.DMA for _ in range(NBUF)],
            [pltpu.SemaphoreType.DMA for _ in range(NBUF)],
            pltpu.SemaphoreType.DMA,
            pltpu.SemaphoreType.DMA,
        ],
    )
    def agg(h_hbm, src_hbm, dst_hbm, zeros_hbm, out_hbm,
            src_v, dst_v, rows, aggr_sh, sem_g, sem_s, sem_i, sem_z):
        c = lax.axis_index("c")
        s = lax.axis_index("s")
        w = c * NS + s
        zcp = pltpu.async_copy(zeros_hbm.at[pl.ds(0, nper)],
                               aggr_sh.at[pl.ds(s * nper, nper)], sem_z)

        @pl.when(s == 0)
        def _():
            pltpu.sync_copy(zeros_hbm.at[pl.ds(0, ntail)],
                            aggr_sh.at[pl.ds(NS * nper, ntail)])

        def gather(j, b, u):
            pltpu.async_copy(h_hbm.at[src_v[u].at[j]], rows[b], sem_g[b])

        def gwait(j, b, u):
            pltpu.make_async_copy(h_hbm.at[src_v[u].at[j]], rows[b],
                                  sem_g[b]).wait()

        def scat(j, b, u):
            pltpu.async_copy(rows[b], aggr_sh.at[dst_v[u].at[j]], sem_s[b],
                             add=True)

        def swait(j, b, u):
            pltpu.make_async_copy(rows[b], aggr_sh.at[dst_v[u].at[j]],
                                  sem_s[b]).wait()

        def stage(p, u):
            pltpu.async_copy(src_hbm.at[w, pl.ds(p * ppc, ppc)], src_v[u],
                             sem_i)
            pltpu.async_copy(dst_hbm.at[w, pl.ds(p * ppc, ppc)], dst_v[u],
                             sem_i)

        def stage_wait(p, u):
            pltpu.make_async_copy(src_hbm.at[w, pl.ds(p * ppc, ppc)],
                                  src_v[u], sem_i).wait()
            pltpu.make_async_copy(dst_hbm.at[w, pl.ds(p * ppc, ppc)],
                                  dst_v[u], sem_i).wait()

        stage(0, 0)
        zcp.wait()
        plsc.subcore_barrier()
        stage_wait(0, 0)
        for p in range(npass):
            u = p % 2

            def body(t, carry, u=u):
                j0 = NBUF * t
                for b in range(NBUF):
                    gwait(j0 + b, b, u)
                    scat(j0 + b, b, u)
                for b in range(NBUF):
                    swait(j0 + b, b, u)
                    gather(j0 + NBUF + b, b, u)
                return carry

            for b in range(NBUF):
                gather(b, b, u)
            if p + 1 < npass:
                stage(p + 1, 1 - u)
            lax.fori_loop(0, ppc // NBUF - 1, body, 0)
            jl = ppc - NBUF
            for b in range(NBUF):
                gwait(jl + b, b, u)
                scat(jl + b, b, u)
            for b in range(NBUF):
                swait(jl + b, b, u)
            if p + 1 < npass:
                stage_wait(p + 1, 1 - u)

        plsc.subcore_barrier()
        pltpu.sync_copy(aggr_sh.at[pl.ds(s * nper, nper)],
                        out_hbm.at[c, pl.ds(s * nper, nper)])

        @pl.when(s == 0)
        def _():
            pltpu.sync_copy(aggr_sh.at[pl.ds(NS * nper, ntail)],
                            out_hbm.at[c, pl.ds(NS * nper, ntail)])

    return agg


def _embed_body(x_ref, ec_ref, o_ref):
    bn = x_ref.shape[0]
    io = lax.broadcasted_iota(jnp.int32, (bn, 2 * 128), 1)
    a_col = x_ref[:, 0:1]
    c_col = x_ref[:, 1:2]
    step = jnp.minimum(io // 128, 1)
    target = a_col * (1 - step) + (c_col + 128) * step
    diff = (target - io).astype(jnp.float32)
    oh = jnp.maximum(1.0 - jnp.abs(diff), 0.0)
    o_ref[...] = jnp.dot(oh, ec_ref[...], preferred_element_type=jnp.float32)


def _layer_body(h_ref, p_ref, ep_ref, w1_ref, b1_ref, w2_ref, b2_ref, o_ref):
    z = h_ref[...] * ep_ref[...] + p_ref[0] + p_ref[1]
    y = jnp.maximum(
        jnp.dot(z, w1_ref[...], preferred_element_type=jnp.float32) + b1_ref[...], 0.0)
    o = jnp.maximum(
        jnp.dot(y, w2_ref[...], preferred_element_type=jnp.float32) + b2_ref[...], 0.0)
    o_ref[...] = o


def _final_body(h_ref, p_ref, b_ref, ep_ref, w1_ref, b1_ref, w2_ref, b2_ref,
                node_ref, graph_ref, acc_sum, acc_cnt):
    i = pl.program_id(0)
    nblocks = pl.num_programs(0)
    z = h_ref[...] * ep_ref[...] + p_ref[0] + p_ref[1]
    y = jnp.maximum(
        jnp.dot(z, w1_ref[...], preferred_element_type=jnp.float32) + b1_ref[...], 0.0)
    o = jnp.maximum(
        jnp.dot(y, w2_ref[...], preferred_element_type=jnp.float32) + b2_ref[...], 0.0)
    node_ref[...] = o

    bn, d = o.shape
    io = lax.broadcasted_iota(jnp.int32, (bn, G), 1)
    oh = jnp.maximum(1.0 - jnp.abs((b_ref[...] - io).astype(jnp.float32)), 0.0)
    dn = (((0,), (0,)), ((), ()))
    psum = lax.dot_general(oh, o, dn, preferred_element_type=jnp.float32)
    pcnt = lax.dot_general(oh, jnp.ones((bn, d), jnp.float32), dn,
                           preferred_element_type=jnp.float32)

    @pl.when(i == 0)
    def _():
        acc_sum[...] = jnp.zeros_like(acc_sum)
        acc_cnt[...] = jnp.zeros_like(acc_cnt)

    acc_sum[...] += psum
    acc_cnt[...] += pcnt

    @pl.when(i == nblocks - 1)
    def _():
        graph_ref[...] = acc_sum[...] / jnp.maximum(acc_cnt[...], 1.0)


def kernel(x, edge_index, batch, atom_emb, chir_emb,
           W1, b1, gamma, beta, run_mean, run_var, W2, b2, eps):
    N = x.shape[0]
    E = edge_index.shape[1]
    L, D, _ = W1.shape
    HALF = atom_emb.shape[1]
    BN = 10000
    K = 50

    f32 = jnp.float32

    scale = gamma / jnp.sqrt(run_var + 1e-5)
    w1t = jnp.transpose(W1 * scale[:, :, None], (0, 2, 1))
    b1f = (scale * (b1 - run_mean) + beta).reshape(L, 1, D)
    w2t = jnp.transpose(W2, (0, 2, 1))
    b2f = b2.reshape(L, 1, D)
    epb = jnp.broadcast_to((1.0 + eps).reshape(L, 1, 1), (L, 1, D)).astype(f32)

    ec = jnp.zeros((2 * 128, D), f32)
    ec = ec.at[:atom_emb.shape[0], :HALF].set(atom_emb)
    ec = ec.at[128:128 + chir_emb.shape[0], HALF:].set(chir_emb)

    W = NC * NS
    src2d = edge_index[0].reshape(W, E // (W * K), K)
    dst2d = edge_index[1].reshape(W, E // (W * K), K)
    zeros_nd = jnp.zeros((((N // NS) // 8) * 8, D), f32)
    batch_col = batch.reshape(N, 1)

    nblk = N // BN

    h = pl.pallas_call(
        _embed_body,
        grid=(nblk,),
        in_specs=[pl.BlockSpec((BN, 2), lambda i: (i, 0)),
                  pl.BlockSpec((2 * 128, D), lambda i: (0, 0))],
        out_specs=pl.BlockSpec((BN, D), lambda i: (i, 0)),
        out_shape=jax.ShapeDtypeStruct((N, D), f32),
    )(x, ec)

    sc_agg = _make_sc_agg(N, E, D, K)

    wspec = pl.BlockSpec((D, D), lambda i: (0, 0))
    bspec = pl.BlockSpec((1, D), lambda i: (0, 0))
    hspec = pl.BlockSpec((BN, D), lambda i: (i, 0))
    pspec = pl.BlockSpec((NC, BN, D), lambda i: (0, i, 0))

    for i in range(L - 1):
        parts = sc_agg(h, src2d, dst2d, zeros_nd)
        h = pl.pallas_call(
            _layer_body,
            grid=(nblk,),
            in_specs=[hspec, pspec, bspec, wspec, bspec, wspec, bspec],
            out_specs=hspec,
            out_shape=jax.ShapeDtypeStruct((N, D), f32),
        )(h, parts, epb[i], w1t[i], b1f[i], w2t[i], b2f[i])

    i = L - 1
    parts = sc_agg(h, src2d, dst2d, zeros_nd)
    node_feats, graph_feats = pl.pallas_call(
        _final_body,
        grid=(nblk,),
        in_specs=[hspec, pspec, pl.BlockSpec((BN, 1), lambda i: (i, 0)),
                  bspec, wspec, bspec, wspec, bspec],
        out_specs=[hspec, pl.BlockSpec((G, D), lambda i: (0, 0))],
        out_shape=[jax.ShapeDtypeStruct((N, D), f32),
                   jax.ShapeDtypeStruct((G, D), f32)],
        scratch_shapes=[pltpu.VMEM((G, D), f32), pltpu.VMEM((G, D), f32)],
    )(h, parts, batch_col, epb[i], w1t[i], b1f[i], w2t[i], b2f[i])

    return (graph_feats, node_feats)


# speedup vs baseline: 1.0090x; 1.0090x over previous
import functools

import jax
import jax.numpy as jnp
from jax import lax
from jax.experimental import pallas as pl
from jax.experimental.pallas import tpu as pltpu
from jax.experimental.pallas import tpu_sc as plsc

NC = 2
NS = 16
G = 256


def _make_sc_agg(N, E, D, K):
    W = NC * NS
    epw = E // W
    nchunk = epw // K
    npass = 5
    ppc = nchunk // npass
    NBUF = 4
    nper = ((N // NS) // 8) * 8
    ntail = N - NS * nper
    mesh = plsc.VectorSubcoreMesh(core_axis_name="c", subcore_axis_name="s")

    @functools.partial(
        pl.kernel,
        out_type=jax.ShapeDtypeStruct((NC, N, D), jnp.float32),
        mesh=mesh,
        scratch_types=[
            [pltpu.VMEM((ppc, K), jnp.int32) for _ in range(2)],
            [pltpu.VMEM((ppc, K), jnp.int32) for _ in range(2)],
            [pltpu.VMEM((K, D), jnp.float32) for _ in range(NBUF)],
            pltpu.VMEM_SHARED((N, D), jnp.float32),
            [pltpu.SemaphoreType.DMA for _ in range(NBUF)],
            [pltpu.SemaphoreType.DMA for _ in range(NBUF)],
            pltpu.SemaphoreType.DMA,
            pltpu.SemaphoreType.DMA,
        ],
    )
    def agg(h_hbm, src_hbm, dst_hbm, zeros_hbm, out_hbm,
            src_v, dst_v, rows, aggr_sh, sem_g, sem_s, sem_i, sem_z):
        c = lax.axis_index("c")
        s = lax.axis_index("s")
        w = c * NS + s
        zcp = pltpu.async_copy(zeros_hbm.at[pl.ds(0, nper)],
                               aggr_sh.at[pl.ds(s * nper, nper)], sem_z)

        @pl.when(s == 0)
        def _():
            pltpu.sync_copy(zeros_hbm.at[pl.ds(0, ntail)],
                            aggr_sh.at[pl.ds(NS * nper, ntail)])

        def gather(j, b, u):
            pltpu.async_copy(h_hbm.at[src_v[u].at[j]], rows[b], sem_g[b])

        def gwait(j, b, u):
            pltpu.make_async_copy(h_hbm.at[src_v[u].at[j]], rows[b],
                                  sem_g[b]).wait()

        def scat(j, b, u):
            pltpu.async_copy(rows[b], aggr_sh.at[dst_v[u].at[j]], sem_s[b],
                             add=True)

        def swait(j, b, u):
            pltpu.make_async_copy(rows[b], aggr_sh.at[dst_v[u].at[j]],
                                  sem_s[b]).wait()

        def stage(p, u):
            pltpu.async_copy(src_hbm.at[w, pl.ds(p * ppc, ppc)], src_v[u],
                             sem_i)
            pltpu.async_copy(dst_hbm.at[w, pl.ds(p * ppc, ppc)], dst_v[u],
                             sem_i)

        def stage_wait(p, u):
            pltpu.make_async_copy(src_hbm.at[w, pl.ds(p * ppc, ppc)],
                                  src_v[u], sem_i).wait()
            pltpu.make_async_copy(dst_hbm.at[w, pl.ds(p * ppc, ppc)],
                                  dst_v[u], sem_i).wait()

        stage(0, 0)
        zcp.wait()
        plsc.subcore_barrier()
        stage_wait(0, 0)
        for p in range(npass):
            u = p % 2

            def body(t, carry, u=u):
                j0 = NBUF * t
                for b in range(NBUF):
                    gwait(j0 + b, b, u)
                    scat(j0 + b, b, u)
                for b in range(NBUF):
                    swait(j0 + b, b, u)
                    gather(j0 + NBUF + b, b, u)
                return carry

            for b in range(NBUF):
                gather(b, b, u)
            if p + 1 < npass:
                stage(p + 1, 1 - u)
            lax.fori_loop(0, ppc // NBUF - 1, body, 0)
            jl = ppc - NBUF
            for b in range(NBUF):
                gwait(jl + b, b, u)
                scat(jl + b, b, u)
            for b in range(NBUF):
                swait(jl + b, b, u)
            if p + 1 < npass:
                stage_wait(p + 1, 1 - u)

        plsc.subcore_barrier()
        pltpu.sync_copy(aggr_sh.at[pl.ds(s * nper, nper)],
                        out_hbm.at[c, pl.ds(s * nper, nper)])

        @pl.when(s == 0)
        def _():
            pltpu.sync_copy(aggr_sh.at[pl.ds(NS * nper, ntail)],
                            out_hbm.at[c, pl.ds(NS * nper, ntail)])

    return agg


def _embed_body(x_ref, ec_ref, o_ref):
    bn = x_ref.shape[0]
    io = lax.broadcasted_iota(jnp.int32, (bn, 2 * 128), 1)
    a_col = x_ref[:, 0:1]
    c_col = x_ref[:, 1:2]
    step = jnp.minimum(io // 128, 1)
    target = a_col * (1 - step) + (c_col + 128) * step
    diff = (target - io).astype(jnp.float32)
    oh = jnp.maximum(1.0 - jnp.abs(diff), 0.0)
    o_ref[...] = jnp.dot(oh, ec_ref[...], preferred_element_type=jnp.float32)


def _layer_body(h_ref, p_ref, ep_ref, w1_ref, b1_ref, w2_ref, b2_ref, o_ref):
    z = h_ref[...] * ep_ref[...] + p_ref[0] + p_ref[1]
    y = jnp.maximum(
        jnp.dot(z, w1_ref[...], preferred_element_type=jnp.float32) + b1_ref[...], 0.0)
    o = jnp.maximum(
        jnp.dot(y, w2_ref[...], preferred_element_type=jnp.float32) + b2_ref[...], 0.0)
    o_ref[...] = o


def _final_body(h_ref, p_ref, b_ref, ep_ref, w1_ref, b1_ref, w2_ref, b2_ref,
                node_ref, graph_ref, acc_sum, acc_cnt):
    i = pl.program_id(0)
    nblocks = pl.num_programs(0)
    z = h_ref[...] * ep_ref[...] + p_ref[0] + p_ref[1]
    y = jnp.maximum(
        jnp.dot(z, w1_ref[...], preferred_element_type=jnp.float32) + b1_ref[...], 0.0)
    o = jnp.maximum(
        jnp.dot(y, w2_ref[...], preferred_element_type=jnp.float32) + b2_ref[...], 0.0)
    node_ref[...] = o

    bn, d = o.shape
    io = lax.broadcasted_iota(jnp.int32, (bn, G), 1)
    oh = jnp.maximum(1.0 - jnp.abs((b_ref[...] - io).astype(jnp.float32)), 0.0)
    dn = (((0,), (0,)), ((), ()))
    psum = lax.dot_general(oh, o, dn, preferred_element_type=jnp.float32)
    pcnt = lax.dot_general(oh, jnp.ones((bn, d), jnp.float32), dn,
                           preferred_element_type=jnp.float32)

    @pl.when(i == 0)
    def _():
        acc_sum[...] = jnp.zeros_like(acc_sum)
        acc_cnt[...] = jnp.zeros_like(acc_cnt)

    acc_sum[...] += psum
    acc_cnt[...] += pcnt

    @pl.when(i == nblocks - 1)
    def _():
        graph_ref[...] = acc_sum[...] / jnp.maximum(acc_cnt[...], 1.0)


def kernel(x, edge_index, batch, atom_emb, chir_emb,
           W1, b1, gamma, beta, run_mean, run_var, W2, b2, eps):
    N = x.shape[0]
    E = edge_index.shape[1]
    L, D, _ = W1.shape
    HALF = atom_emb.shape[1]
    BN = 5000
    K = 50

    f32 = jnp.float32

    scale = gamma / jnp.sqrt(run_var + 1e-5)
    w1t = jnp.transpose(W1 * scale[:, :, None], (0, 2, 1))
    b1f = (scale * (b1 - run_mean) + beta).reshape(L, 1, D)
    w2t = jnp.transpose(W2, (0, 2, 1))
    b2f = b2.reshape(L, 1, D)
    epb = jnp.broadcast_to((1.0 + eps).reshape(L, 1, 1), (L, 1, D)).astype(f32)

    ec = jnp.zeros((2 * 128, D), f32)
    ec = ec.at[:atom_emb.shape[0], :HALF].set(atom_emb)
    ec = ec.at[128:128 + chir_emb.shape[0], HALF:].set(chir_emb)

    W = NC * NS
    src2d = edge_index[0].reshape(W, E // (W * K), K)
    dst2d = edge_index[1].reshape(W, E // (W * K), K)
    zeros_nd = jnp.zeros((((N // NS) // 8) * 8, D), f32)
    batch_col = batch.reshape(N, 1)

    nblk = N // BN

    h = pl.pallas_call(
        _embed_body,
        grid=(nblk,),
        in_specs=[pl.BlockSpec((BN, 2), lambda i: (i, 0)),
                  pl.BlockSpec((2 * 128, D), lambda i: (0, 0))],
        out_specs=pl.BlockSpec((BN, D), lambda i: (i, 0)),
        out_shape=jax.ShapeDtypeStruct((N, D), f32),
    )(x, ec)

    sc_agg = _make_sc_agg(N, E, D, K)

    wspec = pl.BlockSpec((D, D), lambda i: (0, 0))
    bspec = pl.BlockSpec((1, D), lambda i: (0, 0))
    hspec = pl.BlockSpec((BN, D), lambda i: (i, 0))
    pspec = pl.BlockSpec((NC, BN, D), lambda i: (0, i, 0))

    for i in range(L - 1):
        parts = sc_agg(h, src2d, dst2d, zeros_nd)
        h = pl.pallas_call(
            _layer_body,
            grid=(nblk,),
            in_specs=[hspec, pspec, bspec, wspec, bspec, wspec, bspec],
            out_specs=hspec,
            out_shape=jax.ShapeDtypeStruct((N, D), f32),
        )(h, parts, epb[i], w1t[i], b1f[i], w2t[i], b2f[i])

    i = L - 1
    parts = sc_agg(h, src2d, dst2d, zeros_nd)
    node_feats, graph_feats = pl.pallas_call(
        _final_body,
        grid=(nblk,),
        in_specs=[hspec, pspec, pl.BlockSpec((BN, 1), lambda i: (i, 0)),
                  bspec, wspec, bspec, wspec, bspec],
        out_specs=[hspec, pl.BlockSpec((G, D), lambda i: (0, 0))],
        out_shape=[jax.ShapeDtypeStruct((N, D), f32),
                   jax.ShapeDtypeStruct((G, D), f32)],
        scratch_shapes=[pltpu.VMEM((G, D), f32), pltpu.VMEM((G, D), f32)],
    )(h, parts, batch_col, epb[i], w1t[i], b1f[i], w2t[i], b2f[i])

    return (graph_feats, node_feats)
